# baseline (speedup 1.0000x reference)
import functools

import jax
import jax.numpy as jnp
from jax import lax
from jax.experimental import pallas as pl
from jax.experimental.pallas import tpu as pltpu
from jax.experimental.pallas import tpu_sc as plsc

_N = 10000
_E = 320000
_DHID = 128

_NC = 2
_NS = 16
_NW = _NC * _NS
_EPW = _E // _NW
_CH = 80
_NCHUNKS = _EPW // _CH
_NPAD = 10240
_RPT = _NPAD // _NS
_CHD = 10000
_NCHUNKS_D = _EPW // _CHD


def _make_prop(d, ch, nbuf, npass=1, tc_tiling=True):
    nch = _EPW // ch
    assert nch * ch == _EPW and ch % 16 == 0 and nch >= 2 * nbuf - 1
    h = ((nch - nbuf) % nbuf) + 1
    nblk = (nch - nbuf + 1 - h) // nbuf
    mesh = plsc.VectorSubcoreMesh(core_axis_name="c", subcore_axis_name="s")

    @functools.partial(
        pl.kernel, mesh=mesh,
        out_type=[jax.ShapeDtypeStruct((_NC * _NPAD, d), jnp.float32)
                  for _ in range(npass)],
        scratch_types=[
            pltpu.VMEM((_EPW,), jnp.int32),
            pltpu.VMEM((_EPW,), jnp.int32),
        ] + [pltpu.VMEM((ch,), jnp.int32) for _ in range(nbuf)]
          + [pltpu.VMEM((ch, d), jnp.float32) for _ in range(nbuf)]
          + [pltpu.VMEM_SHARED((_NPAD, d), jnp.float32)]
          + [pltpu.SemaphoreType.DMA for _ in range(2 * nbuf)],
        compiler_params=pltpu.CompilerParams(use_tc_tiling_on_sc=tc_tiling),
    )
    def k(*args):
        srcs = args[:npass]
        rows, cols, zrows = args[npass:npass + 3]
        outs = args[npass + 3:2 * npass + 3]
        rall, call = args[2 * npass + 3:2 * npass + 5]
        rest = args[2 * npass + 5:]
        cx = rest[:nbuf]
        rb = rest[nbuf:2 * nbuf]
        acc = rest[2 * nbuf]
        gsem = rest[2 * nbuf + 1:3 * nbuf + 1]
        ssem = rest[3 * nbuf + 1:]

        c = lax.axis_index("c")
        s = lax.axis_index("s")
        wid = s * _NC + c
        base = wid * _EPW

        pltpu.async_copy(rows.at[pl.ds(base, _EPW)], rall, gsem[0])
        pltpu.async_copy(cols.at[pl.ds(base, _EPW)], call, gsem[1])
        pltpu.sync_copy(zrows, acc.at[pl.ds(s * _RPT, _RPT)])
        pltpu.make_async_copy(rows.at[pl.ds(base, _EPW)], rall,
                              gsem[0]).wait()
        pltpu.make_async_copy(cols.at[pl.ds(base, _EPW)], call,
                              gsem[1]).wait()

        def fill_cidx(i, b):
            def mv(j, carry):
                cx[b][pl.ds(j * 16, 16)] = call[pl.ds(i * ch + j * 16, 16)]
                return carry
            lax.fori_loop(0, ch // 16, mv, 0)

        def start_scatter(b):
            pltpu.async_copy(rb[b], acc.at[cx[b]], ssem[b], add=True)

        def wait_scatter(b):
            pltpu.make_async_copy(rb[b], acc.at[cx[b]], ssem[b]).wait()

        def run_pass(src, out):
            def start_gather(i, b):
                pltpu.async_copy(src.at[rall.at[pl.ds(i * ch, ch)]], rb[b],
                                 gsem[b])

            def wait_gather(i, b):
                pltpu.make_async_copy(src.at[rall.at[pl.ds(i * ch, ch)]],
                                      rb[b], gsem[b]).wait()

            def sub(i, b, bn, first):
                if not first:
                    wait_scatter(bn)
                start_gather(i + nbuf - 1, bn)
                fill_cidx(i + nbuf - 1, bn)
                wait_gather(i, b)
                start_scatter(b)

            plsc.subcore_barrier()

            for j in range(nbuf - 1):
                start_gather(j, j)
                fill_cidx(j, j)

            for i in range(h):
                sub(i, i % nbuf, (i - 1) % nbuf, first=(i == 0))

            def blk(g, carry):
                for t in range(nbuf):
                    i = h + nbuf * g + t
                    sub(i, (h + t) % nbuf, (h + t - 1) % nbuf, first=False)
                return carry

            lax.fori_loop(0, nblk, blk, 0)

            for i in range(nch - nbuf + 1, nch):
                wait_gather(i, i % nbuf)
                start_scatter(i % nbuf)

            for b in range(nbuf):
                wait_scatter(b)

            plsc.subcore_barrier()
            pltpu.sync_copy(acc.at[pl.ds(s * _RPT, _RPT)],
                            out.at[pl.ds(c * _NPAD + s * _RPT, _RPT)])

        for p in range(npass):
            if p > 0:
                pltpu.sync_copy(zrows, acc.at[pl.ds(s * _RPT, _RPT)])
            run_pass(srcs[p], outs[p])

    return k


def _make_deg():
    mesh = plsc.VectorSubcoreMesh(core_axis_name="c", subcore_axis_name="s")

    @functools.partial(
        pl.kernel, mesh=mesh,
        out_type=jax.ShapeDtypeStruct((_NW, _NPAD), jnp.float32),
        scratch_types=[
            pltpu.VMEM((_CHD,), jnp.int32),
            pltpu.VMEM((_NPAD,), jnp.float32),
        ],
        compiler_params=pltpu.CompilerParams(needs_layout_passes=False),
    )
    def k(cols, out, cidx, hist):
        c = lax.axis_index("c")
        s = lax.axis_index("s")
        wid = s * _NC + c

        z16 = jnp.zeros((16,), jnp.float32)

        def zero(i, carry):
            hist[pl.ds(i * 16, 16)] = z16
            return carry

        lax.fori_loop(0, _NPAD // 16, zero, 0)

        def chunk(i, carry):
            base = wid * _EPW + i * _CHD
            pltpu.sync_copy(cols.at[pl.ds(base, _CHD)], cidx)

            def vec(j, carry2):
                idx16 = cidx[pl.ds(j * 16, 16)]
                cnt, last = plsc.scan_count(idx16)
                plsc.addupdate_scatter(
                    hist, [idx16], cnt.astype(jnp.float32), mask=last)
                return carry2

            return lax.fori_loop(0, _CHD // 16, vec, carry)

        lax.fori_loop(0, _NCHUNKS_D, chunk, 0)
        pltpu.sync_copy(hist, out.at[wid])

    return k


_prop64x2 = _make_prop(64, 400, 2, npass=2, tc_tiling=False)
_prop8 = _make_prop(8, 2000, 2, tc_tiling=False)
_deg_pass = _make_deg()


def _dinv_from(deg_part_ref):
    ones32 = jnp.ones((_NW, 1), jnp.float32)
    deg = lax.dot_general(deg_part_ref[...], ones32,
                          (((0,), (0,)), ((), ())),
                          preferred_element_type=jnp.float32)
    return lax.rsqrt(deg[: _N, :] + 1.0)


def _tc1_body(dp, f, w, g1a, g1b):
    dinv = _dinv_from(dp)
    g1 = jnp.dot(f[...], w[...], preferred_element_type=jnp.float32) * dinv
    g1a[...] = g1[:, :64]
    g1b[...] = g1[:, 64:]


def _tc2_body(dp, sa, sb, g1a, g1b, b1, w2p, g2):
    dinv = _dinv_from(dp)
    agga = sa[: _N, :] + sa[_NPAD : _NPAD + _N, :] + g1a[...]
    aggb = sb[: _N, :] + sb[_NPAD : _NPAD + _N, :] + g1b[...]
    x1a = jnp.maximum(agga * dinv + b1[:, :64], 0.0)
    x1b = jnp.maximum(aggb * dinv + b1[:, 64:], 0.0)
    g2[...] = (jnp.dot(x1a, w2p[:64], preferred_element_type=jnp.float32)
               + jnp.dot(x1b, w2p[64:], preferred_element_type=jnp.float32)
               ) * dinv


def _tc3_body(dp, scat2, g2, b2p, out):
    dinv = _dinv_from(dp)
    z = (scat2[: _N, :] + scat2[_NPAD : _NPAD + _N, :] + g2[...]) * dinv
    out[...] = z[:, :3] + b2p[...]


def kernel(features, edges, edges2, edge_features, W1, b1, W2, b2):
    del edges2, edge_features
    rows = edges[0]
    cols = edges[1]

    zeros64 = jnp.zeros((_RPT, 64), jnp.float32)
    zeros8 = jnp.zeros((_RPT, 8), jnp.float32)
    b1_2d = b1.reshape(1, _DHID)
    b2_2d = b2.reshape(1, 3)
    w2p = jnp.zeros((_DHID, 8), jnp.float32).at[:, :3].set(W2)

    deg_part = _deg_pass(cols)

    g1a, g1b = pl.pallas_call(
        _tc1_body,
        out_shape=[jax.ShapeDtypeStruct((_N, 64), jnp.float32)] * 2,
    )(deg_part, features, W1)

    scat_a, scat_b = _prop64x2(g1a, g1b, rows, cols, zeros64)

    g2 = pl.pallas_call(
        _tc2_body,
        out_shape=jax.ShapeDtypeStruct((_N, 8), jnp.float32),
    )(deg_part, scat_a, scat_b, g1a, g1b, b1_2d, w2p)

    (scat2,) = _prop8(g2, rows, cols, zeros8)

    out = pl.pallas_call(
        _tc3_body,
        out_shape=jax.ShapeDtypeStruct((_N, 3), jnp.float32),
    )(deg_part, scat2, g2, b2_2d)

    return out


# speedup vs baseline: 1.0862x; 1.0862x over previous
import functools

import jax
import jax.numpy as jnp
from jax import lax
from jax.experimental import pallas as pl
from jax.experimental.pallas import tpu as pltpu
from jax.experimental.pallas import tpu_sc as plsc

_N = 10000
_E = 320000
_DHID = 128

_NC = 2
_NS = 16
_NW = _NC * _NS
_EPW = _E // _NW
_CH = 80
_NCHUNKS = _EPW // _CH
_NPAD = 10240
_RPT = _NPAD // _NS
_CHD = 10000
_NCHUNKS_D = _EPW // _CHD


def _make_prop(d, ch, nbuf, npass=1, tc_tiling=True):
    nch = _EPW // ch
    assert nch * ch == _EPW and ch % 16 == 0 and nch >= 2 * nbuf - 1
    h = ((nch - nbuf) % nbuf) + 1
    nblk = (nch - nbuf + 1 - h) // nbuf
    mesh = plsc.VectorSubcoreMesh(core_axis_name="c", subcore_axis_name="s")

    @functools.partial(
        pl.kernel, mesh=mesh,
        out_type=[jax.ShapeDtypeStruct((_NC * _NPAD, d), jnp.float32)
                  for _ in range(npass)],
        scratch_types=[
            pltpu.VMEM((_EPW,), jnp.int32),
            pltpu.VMEM((_EPW,), jnp.int32),
        ] + [pltpu.VMEM((ch,), jnp.int32) for _ in range(nbuf)]
          + [pltpu.VMEM((ch, d), jnp.float32) for _ in range(nbuf)]
          + [pltpu.VMEM_SHARED((_NPAD, d), jnp.float32)]
          + [pltpu.SemaphoreType.DMA for _ in range(2 * nbuf)],
        compiler_params=pltpu.CompilerParams(use_tc_tiling_on_sc=tc_tiling),
    )
    def k(*args):
        srcs = args[:npass]
        rows, cols, zrows = args[npass:npass + 3]
        outs = args[npass + 3:2 * npass + 3]
        rall, call = args[2 * npass + 3:2 * npass + 5]
        rest = args[2 * npass + 5:]
        cx = rest[:nbuf]
        rb = rest[nbuf:2 * nbuf]
        acc = rest[2 * nbuf]
        gsem = rest[2 * nbuf + 1:3 * nbuf + 1]
        ssem = rest[3 * nbuf + 1:]

        c = lax.axis_index("c")
        s = lax.axis_index("s")
        wid = s * _NC + c
        base = wid * _EPW

        pltpu.async_copy(rows.at[pl.ds(base, _EPW)], rall, gsem[0])
        pltpu.async_copy(cols.at[pl.ds(base, _EPW)], call, gsem[1])
        pltpu.sync_copy(zrows, acc.at[pl.ds(s * _RPT, _RPT)])
        pltpu.make_async_copy(rows.at[pl.ds(base, _EPW)], rall,
                              gsem[0]).wait()
        pltpu.make_async_copy(cols.at[pl.ds(base, _EPW)], call,
                              gsem[1]).wait()

        def fill_cidx(i, b):
            def mv(j, carry):
                cx[b][pl.ds(j * 16, 16)] = call[pl.ds(i * ch + j * 16, 16)]
                return carry
            lax.fori_loop(0, ch // 16, mv, 0)

        def start_scatter(b):
            pltpu.async_copy(rb[b], acc.at[cx[b]], ssem[b], add=True)

        def wait_scatter(b):
            pltpu.make_async_copy(rb[b], acc.at[cx[b]], ssem[b]).wait()

        def run_pass(src, out):
            def start_gather(i, b):
                pltpu.async_copy(src.at[rall.at[pl.ds(i * ch, ch)]], rb[b],
                                 gsem[b])

            def wait_gather(i, b):
                pltpu.make_async_copy(src.at[rall.at[pl.ds(i * ch, ch)]],
                                      rb[b], gsem[b]).wait()

            def sub(i, b, bn, first):
                if not first:
                    wait_scatter(bn)
                start_gather(i + nbuf - 1, bn)
                fill_cidx(i + nbuf - 1, bn)
                wait_gather(i, b)
                start_scatter(b)

            plsc.subcore_barrier()

            for j in range(nbuf - 1):
                start_gather(j, j)
                fill_cidx(j, j)

            for i in range(h):
                sub(i, i % nbuf, (i - 1) % nbuf, first=(i == 0))

            def blk(g, carry):
                for t in range(nbuf):
                    i = h + nbuf * g + t
                    sub(i, (h + t) % nbuf, (h + t - 1) % nbuf, first=False)
                return carry

            lax.fori_loop(0, nblk, blk, 0)

            for i in range(nch - nbuf + 1, nch):
                wait_gather(i, i % nbuf)
                start_scatter(i % nbuf)

            for b in range(nbuf):
                wait_scatter(b)

            plsc.subcore_barrier()
            pltpu.sync_copy(acc.at[pl.ds(s * _RPT, _RPT)],
                            out.at[pl.ds(c * _NPAD + s * _RPT, _RPT)])

        for p in range(npass):
            if p > 0:
                pltpu.sync_copy(zrows, acc.at[pl.ds(s * _RPT, _RPT)])
            run_pass(srcs[p], outs[p])

    return k


def _make_deg():
    mesh = plsc.VectorSubcoreMesh(core_axis_name="c", subcore_axis_name="s")

    @functools.partial(
        pl.kernel, mesh=mesh,
        out_type=jax.ShapeDtypeStruct((_NW, _NPAD), jnp.float32),
        scratch_types=[
            pltpu.VMEM((_CHD,), jnp.int32),
            pltpu.VMEM((_NPAD,), jnp.float32),
        ],
        compiler_params=pltpu.CompilerParams(needs_layout_passes=False),
    )
    def k(cols, out, cidx, hist):
        c = lax.axis_index("c")
        s = lax.axis_index("s")
        wid = s * _NC + c

        z16 = jnp.zeros((16,), jnp.float32)

        def zero(i, carry):
            hist[pl.ds(i * 16, 16)] = z16
            return carry

        lax.fori_loop(0, _NPAD // 16, zero, 0)

        def chunk(i, carry):
            base = wid * _EPW + i * _CHD
            pltpu.sync_copy(cols.at[pl.ds(base, _CHD)], cidx)

            def vec(j, carry2):
                idx16 = cidx[pl.ds(j * 16, 16)]
                cnt, last = plsc.scan_count(idx16)
                plsc.addupdate_scatter(
                    hist, [idx16], cnt.astype(jnp.float32), mask=last)
                return carry2

            return lax.fori_loop(0, _CHD // 16, vec, carry)

        lax.fori_loop(0, _NCHUNKS_D, chunk, 0)
        pltpu.sync_copy(hist, out.at[wid])

    return k


_prop128 = _make_prop(_DHID, _CH, 2)
_prop8 = _make_prop(8, 2000, 3, tc_tiling=False)
_deg_pass = _make_deg()


def _dinv_from(deg_part_ref):
    ones32 = jnp.ones((_NW, 1), jnp.float32)
    deg = lax.dot_general(deg_part_ref[...], ones32,
                          (((0,), (0,)), ((), ())),
                          preferred_element_type=jnp.float32)
    return lax.rsqrt(deg[: _N, :] + 1.0)


def _tc1_body(dp, f, w, g1):
    dinv = _dinv_from(dp)
    g1[...] = jnp.dot(f[...], w[...], preferred_element_type=jnp.float32) * dinv


def _tc2_body(dp, scat1, g1, b1, w2p, g2):
    dinv = _dinv_from(dp)
    agg = scat1[: _N, :] + scat1[_NPAD : _NPAD + _N, :] + g1[...]
    x1 = jnp.maximum(agg * dinv + b1[...], 0.0)
    g2[...] = jnp.dot(x1, w2p[...], preferred_element_type=jnp.float32) * dinv


def _tc3_body(dp, scat2, g2, b2p, out):
    dinv = _dinv_from(dp)
    z = (scat2[: _N, :] + scat2[_NPAD : _NPAD + _N, :] + g2[...]) * dinv
    out[...] = z[:, :3] + b2p[...]


def kernel(features, edges, edges2, edge_features, W1, b1, W2, b2):
    del edges2, edge_features
    rows = edges[0]
    cols = edges[1]

    zeros128 = jnp.zeros((_RPT, _DHID), jnp.float32)
    zeros8 = jnp.zeros((_RPT, 8), jnp.float32)
    b1_2d = b1.reshape(1, _DHID)
    b2_2d = b2.reshape(1, 3)
    w2p = jnp.zeros((_DHID, 8), jnp.float32).at[:, :3].set(W2)

    deg_part = _deg_pass(cols)

    g1 = pl.pallas_call(
        _tc1_body,
        out_shape=jax.ShapeDtypeStruct((_N, _DHID), jnp.float32),
    )(deg_part, features, W1)

    (scat1,) = _prop128(g1, rows, cols, zeros128)

    g2 = pl.pallas_call(
        _tc2_body,
        out_shape=jax.ShapeDtypeStruct((_N, 8), jnp.float32),
    )(deg_part, scat1, g1, b1_2d, w2p)

    (scat2,) = _prop8(g2, rows, cols, zeros8)

    out = pl.pallas_call(
        _tc3_body,
        out_shape=jax.ShapeDtypeStruct((_N, 3), jnp.float32),
    )(deg_part, scat2, g2, b2_2d)

    return out
